# baseline (speedup 1.0000x reference)
import functools

import numpy as np

import jax
import jax.numpy as jnp
from jax.experimental import pallas as pl
from jax.experimental.pallas import tpu as pltpu
from jax.sharding import Mesh, PartitionSpec as P


def _dot_f32(a, b):
    return jax.lax.dot_general(
        a, b, (((1,), (0,)), ((), ())),
        precision=jax.lax.Precision.DEFAULT,
        preferred_element_type=jnp.float32)


def _xwt_kernel(x_ref, w_ref, o_ref):
    o_ref[:] = jax.lax.dot_general(
        w_ref[:], jnp.maximum(x_ref[:], 0.0),
        (((0,), (1,)), ((), ())),
        precision=jax.lax.Precision.DEFAULT,
        preferred_element_type=jnp.float32)


def _xw_t_pass(x, w, bm=2048):
    m, k = x.shape
    c = w.shape[1]
    return pl.pallas_call(
        _xwt_kernel,
        grid=(m // bm,),
        in_specs=[
            pl.BlockSpec((bm, k), lambda i: (i, 0)),
            pl.BlockSpec((k, c), lambda i: (0, 0)),
        ],
        out_specs=pl.BlockSpec((c, bm), lambda i: (0, i)),
        out_shape=jax.ShapeDtypeStruct((c, m), jnp.float32),
        compiler_params=pltpu.CompilerParams(
            dimension_semantics=("parallel",)),
    )(x, w)


def _stream_kernel(n_ref, at_ref, o_ref):
    o_ref[:] = jnp.maximum(
        jax.lax.dot_general(
            at_ref[:], n_ref[:],
            (((1,), (1,)), ((), ())),
            precision=jax.lax.Precision.DEFAULT,
            preferred_element_type=jnp.float32), 0.0)


def _stream_t_pass(n, at, bm):
    m, k = n.shape
    c = at.shape[0]
    return pl.pallas_call(
        _stream_kernel,
        grid=(m // bm,),
        in_specs=[
            pl.BlockSpec((bm, k), lambda i: (i, 0)),
            pl.BlockSpec((c, k), lambda i: (0, 0)),
        ],
        out_specs=pl.BlockSpec((c, bm), lambda i: (0, i)),
        out_shape=jax.ShapeDtypeStruct((c, m), jnp.float32),
        compiler_params=pltpu.CompilerParams(
            dimension_semantics=("parallel",)),
    )(n, at)


def _mid_t_kernel(x0at_ref, w_ref, o_ref):
    o_ref[:] = jax.lax.dot_general(
        w_ref[:], x0at_ref[:], (((0,), (0,)), ((), ())),
        precision=jax.lax.Precision.DEFAULT,
        preferred_element_type=jnp.float32)


def _sums_kernel(x0bt_ref, x2t_ref, x1_ref, s0_ref, s2_ref, s1_ref):
    s0_ref[:] = jnp.sum(x0bt_ref[:], axis=1, keepdims=True)
    s2_ref[:] = jnp.sum(x2t_ref[:], axis=1, keepdims=True)
    s1_ref[:] = jnp.sum(jnp.maximum(x1_ref[:], 0.0), axis=0, keepdims=True)


def _final_kernel(s0_ref, s2_ref, s1_ref,
                  w0_ref, b0_ref, w1_ref, b1_ref, w2_ref, b2_ref, o_ref,
                  *, n_nodes, n_edges, n_faces):
    m0 = s0_ref[:] / n_nodes
    m1 = s1_ref[:] / n_edges
    m2 = s2_ref[:] / n_faces
    o_ref[:] = (
        jax.lax.dot_general(m0, w0_ref[:], (((0,), (0,)), ((), ())),
                            preferred_element_type=jnp.float32)
        + b0_ref[:]
        + _dot_f32(m1, w1_ref[:]) + b1_ref[:]
        + jax.lax.dot_general(m2, w2_ref[:], (((0,), (0,)), ((), ())),
                              preferred_element_type=jnp.float32)
        + b2_ref[:])


def kernel(x_0, x_1, neighborhood_0_to_0, neighborhood_1_to_2,
           w00_l0, w12_l0, w00_l1, w12_l1,
           lin0_w, lin0_b, lin1_w, lin1_b, lin2_w, lin2_b):
    n_nodes = x_0.shape[0]
    n_edges = x_1.shape[0]
    n_faces = neighborhood_1_to_2.shape[0]
    ncls = lin0_w.shape[1]

    devs = jax.devices()
    n_shards = 1
    for cand in (8, 4, 2):
        if len(devs) >= cand and n_faces % cand == 0 and n_nodes % cand == 0:
            n_shards = cand
            break
    mesh = Mesh(np.array(devs[:n_shards]), ("d",))

    final = functools.partial(_final_kernel, n_nodes=float(n_nodes),
                              n_edges=float(n_edges), n_faces=float(n_faces))

    def run(x_0, x_1, n00, n12, w00_l0, w00_l1, w12_l1,
            lin0_w, lin0_b, lin1_w, lin1_b, lin2_w, lin2_b):
        a0t = _xw_t_pass(x_0, w00_l0)
        bt = _xw_t_pass(x_1, w12_l1)

        x0at_loc = _stream_t_pass(n00, a0t, bm=256)
        x0at = jax.lax.all_gather(x0at_loc, "d", axis=1, tiled=True)

        a1t = pl.pallas_call(
            _mid_t_kernel,
            out_shape=jax.ShapeDtypeStruct(x0at.shape, jnp.float32),
        )(x0at, w00_l1)

        x0bt_loc = _stream_t_pass(n00, a1t, bm=256)
        x2t_loc = _stream_t_pass(n12, bt, bm=256)

        s0, s2, s1 = pl.pallas_call(
            _sums_kernel,
            out_shape=(
                jax.ShapeDtypeStruct((x0bt_loc.shape[0], 1), jnp.float32),
                jax.ShapeDtypeStruct((x2t_loc.shape[0], 1), jnp.float32),
                jax.ShapeDtypeStruct((1, x_1.shape[1]), jnp.float32),
            ),
        )(x0bt_loc, x2t_loc, x_1)
        s0 = jax.lax.psum(s0, "d")
        s2 = jax.lax.psum(s2, "d")

        out = pl.pallas_call(
            final,
            out_shape=jax.ShapeDtypeStruct((1, ncls), jnp.float32),
        )(s0, s2, s1,
          lin0_w, lin0_b.reshape(1, ncls), lin1_w, lin1_b.reshape(1, ncls),
          lin2_w, lin2_b.reshape(1, ncls))
        return out.reshape(ncls)

    sharded = jax.shard_map(
        run, mesh=mesh,
        in_specs=(P(), P(), P("d", None), P("d", None),
                  P(), P(), P(), P(), P(), P(), P(), P(), P()),
        out_specs=P(),
        check_vma=False,
    )
    return sharded(x_0, x_1, neighborhood_0_to_0, neighborhood_1_to_2,
                   w00_l0, w00_l1, w12_l1,
                   lin0_w, lin0_b, lin1_w, lin1_b, lin2_w, lin2_b)


# speedup vs baseline: 4.0976x; 4.0976x over previous
import functools

import jax
import jax.numpy as jnp
from jax.experimental import pallas as pl
from jax.experimental.pallas import tpu as pltpu


def _dot_f32(a, b):
    return jax.lax.dot_general(
        a, b, (((1,), (0,)), ((), ())),
        precision=jax.lax.Precision.DEFAULT,
        preferred_element_type=jnp.float32)


def _xwt_kernel(x_ref, w_ref, o_ref):
    o_ref[:] = jax.lax.dot_general(
        w_ref[:], jnp.maximum(x_ref[:], 0.0),
        (((0,), (1,)), ((), ())),
        precision=jax.lax.Precision.DEFAULT,
        preferred_element_type=jnp.float32)


def _xw_t_pass(x, w, bm=2048):
    m, k = x.shape
    c = w.shape[1]
    return pl.pallas_call(
        _xwt_kernel,
        grid=(m // bm,),
        in_specs=[
            pl.BlockSpec((bm, k), lambda i: (i, 0)),
            pl.BlockSpec((k, c), lambda i: (0, 0)),
        ],
        out_specs=pl.BlockSpec((c, bm), lambda i: (0, i)),
        out_shape=jax.ShapeDtypeStruct((c, m), jnp.float32),
        compiler_params=pltpu.CompilerParams(
            dimension_semantics=("parallel",)),
    )(x, w)


def _stream_kernel(n_ref, at_ref, o_ref):
    o_ref[:] = jnp.maximum(
        jax.lax.dot_general(
            at_ref[:], n_ref[:],
            (((1,), (1,)), ((), ())),
            precision=jax.lax.Precision.DEFAULT,
            preferred_element_type=jnp.float32), 0.0)


def _stream_t_pass(n, at, bm):
    m, k = n.shape
    c = at.shape[0]
    return pl.pallas_call(
        _stream_kernel,
        grid=(m // bm,),
        in_specs=[
            pl.BlockSpec((bm, k), lambda i: (i, 0)),
            pl.BlockSpec((c, k), lambda i: (0, 0)),
        ],
        out_specs=pl.BlockSpec((c, bm), lambda i: (0, i)),
        out_shape=jax.ShapeDtypeStruct((c, m), jnp.float32),
        compiler_params=pltpu.CompilerParams(
            dimension_semantics=("parallel",)),
    )(n, at)


def _mid_t_kernel(x0at_ref, w_ref, o_ref):
    o_ref[:] = jax.lax.dot_general(
        w_ref[:], x0at_ref[:], (((0,), (0,)), ((), ())),
        precision=jax.lax.Precision.DEFAULT,
        preferred_element_type=jnp.float32)


def _sums_kernel(x0bt_ref, x2t_ref, x1_ref, s0_ref, s2_ref, s1_ref):
    s0_ref[:] = jnp.sum(x0bt_ref[:], axis=1, keepdims=True)
    s2_ref[:] = jnp.sum(x2t_ref[:], axis=1, keepdims=True)
    s1_ref[:] = jnp.sum(jnp.maximum(x1_ref[:], 0.0), axis=0, keepdims=True)


def _final_kernel(s0_ref, s2_ref, s1_ref,
                  w0_ref, b0_ref, w1_ref, b1_ref, w2_ref, b2_ref, o_ref,
                  *, n_nodes, n_edges, n_faces):
    m0 = s0_ref[:] / n_nodes
    m1 = s1_ref[:] / n_edges
    m2 = s2_ref[:] / n_faces
    o_ref[:] = (
        jax.lax.dot_general(m0, w0_ref[:], (((0,), (0,)), ((), ())),
                            preferred_element_type=jnp.float32)
        + b0_ref[:]
        + _dot_f32(m1, w1_ref[:]) + b1_ref[:]
        + jax.lax.dot_general(m2, w2_ref[:], (((0,), (0,)), ((), ())),
                              preferred_element_type=jnp.float32)
        + b2_ref[:])


def kernel(x_0, x_1, neighborhood_0_to_0, neighborhood_1_to_2,
           w00_l0, w12_l0, w00_l1, w12_l1,
           lin0_w, lin0_b, lin1_w, lin1_b, lin2_w, lin2_b):
    n_nodes = x_0.shape[0]
    n_edges = x_1.shape[0]
    n_faces = neighborhood_1_to_2.shape[0]
    ncls = lin0_w.shape[1]

    final = functools.partial(_final_kernel, n_nodes=float(n_nodes),
                              n_edges=float(n_edges), n_faces=float(n_faces))

    a0t = _xw_t_pass(x_0, w00_l0)
    bt = _xw_t_pass(x_1, w12_l1)

    x0at = _stream_t_pass(neighborhood_0_to_0, a0t, bm=256)

    a1t = pl.pallas_call(
        _mid_t_kernel,
        out_shape=jax.ShapeDtypeStruct(x0at.shape, jnp.float32),
    )(x0at, w00_l1)

    x0bt = _stream_t_pass(neighborhood_0_to_0, a1t, bm=256)
    x2t = _stream_t_pass(neighborhood_1_to_2, bt, bm=256)

    s0, s2, s1 = pl.pallas_call(
        _sums_kernel,
        out_shape=(
            jax.ShapeDtypeStruct((x0bt.shape[0], 1), jnp.float32),
            jax.ShapeDtypeStruct((x2t.shape[0], 1), jnp.float32),
            jax.ShapeDtypeStruct((1, x_1.shape[1]), jnp.float32),
        ),
    )(x0bt, x2t, x_1)

    out = pl.pallas_call(
        final,
        out_shape=jax.ShapeDtypeStruct((1, ncls), jnp.float32),
    )(s0, s2, s1,
      lin0_w, lin0_b.reshape(1, ncls), lin1_w, lin1_b.reshape(1, ncls),
      lin2_w, lin2_b.reshape(1, ncls))
    return out.reshape(ncls)


# speedup vs baseline: 4.4219x; 1.0791x over previous
import functools

import jax
import jax.numpy as jnp
from jax.experimental import pallas as pl
from jax.experimental.pallas import tpu as pltpu


def _dot_f32(a, b):
    return jax.lax.dot_general(
        a, b, (((1,), (0,)), ((), ())),
        precision=jax.lax.Precision.DEFAULT,
        preferred_element_type=jnp.float32)


def _wt_xt(w, x):
    return jax.lax.dot_general(
        w, jnp.maximum(x, 0.0), (((0,), (1,)), ((), ())),
        precision=jax.lax.Precision.DEFAULT,
        preferred_element_type=jnp.float32)


def _nt_dot(at, n):
    return jax.lax.dot_general(
        at, n, (((1,), (1,)), ((), ())),
        precision=jax.lax.Precision.DEFAULT,
        preferred_element_type=jnp.float32)


def _stream1_kernel(n_ref, x0_ref, w_ref, o_ref, a0t_ref):
    @pl.when(pl.program_id(0) == 0)
    def _():
        a0t_ref[:] = _wt_xt(w_ref[:], x0_ref[:])
    o_ref[:] = jnp.maximum(_nt_dot(a0t_ref[:], n_ref[:]), 0.0)


def _stream2_kernel(n_ref, x0at_ref, w_ref, o_ref, a1t_ref):
    @pl.when(pl.program_id(0) == 0)
    def _():
        a1t_ref[:] = jax.lax.dot_general(
            w_ref[:], x0at_ref[:], (((0,), (0,)), ((), ())),
            precision=jax.lax.Precision.DEFAULT,
            preferred_element_type=jnp.float32)
    x0bt = jnp.maximum(_nt_dot(a1t_ref[:], n_ref[:]), 0.0)
    o_ref[:] = jnp.sum(x0bt, axis=1, keepdims=True)[None]


def _stream3_kernel(n_ref, x1_ref, w_ref, o_ref, s1_ref, bt_ref):
    @pl.when(pl.program_id(0) == 0)
    def _():
        bt_ref[:] = _wt_xt(w_ref[:], x1_ref[:])
        s1_ref[:] = jnp.sum(jnp.maximum(x1_ref[:], 0.0), axis=0,
                            keepdims=True)
    x2t = jnp.maximum(_nt_dot(bt_ref[:], n_ref[:]), 0.0)
    o_ref[:] = jnp.sum(x2t, axis=1, keepdims=True)[None]


def _final_kernel(s0_ref, s2_ref, s1_ref,
                  w0_ref, b0_ref, w1_ref, b1_ref, w2_ref, b2_ref, o_ref,
                  *, n_nodes, n_edges, n_faces):
    m0 = jnp.sum(s0_ref[:], axis=0) / n_nodes
    m1 = s1_ref[:] / n_edges
    m2 = jnp.sum(s2_ref[:], axis=0) / n_faces
    o_ref[:] = (
        jax.lax.dot_general(m0, w0_ref[:], (((0,), (0,)), ((), ())),
                            preferred_element_type=jnp.float32)
        + b0_ref[:]
        + _dot_f32(m1, w1_ref[:]) + b1_ref[:]
        + jax.lax.dot_general(m2, w2_ref[:], (((0,), (0,)), ((), ())),
                              preferred_element_type=jnp.float32)
        + b2_ref[:])


_N_BUF = 2
_BM = 256


def _n_spec(bm, k):
    return pl.BlockSpec((bm, k), lambda i: (i, 0),
                        pipeline_mode=pl.Buffered(buffer_count=_N_BUF))


def kernel(x_0, x_1, neighborhood_0_to_0, neighborhood_1_to_2,
           w00_l0, w12_l0, w00_l1, w12_l1,
           lin0_w, lin0_b, lin1_w, lin1_b, lin2_w, lin2_b):
    n_nodes, c0 = x_0.shape
    n_edges, c1 = x_1.shape
    n_faces = neighborhood_1_to_2.shape[0]
    c2 = w12_l1.shape[1]
    ncls = lin0_w.shape[1]
    bm = _BM
    g1 = n_nodes // bm
    g3 = n_faces // bm

    params = pltpu.CompilerParams(dimension_semantics=("arbitrary",))

    x0at = pl.pallas_call(
        _stream1_kernel,
        grid=(g1,),
        in_specs=[
            _n_spec(bm, n_nodes),
            pl.BlockSpec((n_nodes, c0), lambda i: (0, 0)),
            pl.BlockSpec((c0, c0), lambda i: (0, 0)),
        ],
        out_specs=pl.BlockSpec((c0, bm), lambda i: (0, i)),
        out_shape=jax.ShapeDtypeStruct((c0, n_nodes), jnp.float32),
        scratch_shapes=[pltpu.VMEM((c0, n_nodes), jnp.float32)],
        compiler_params=params,
    )(neighborhood_0_to_0, x_0, w00_l0)

    s0p = pl.pallas_call(
        _stream2_kernel,
        grid=(g1,),
        in_specs=[
            _n_spec(bm, n_nodes),
            pl.BlockSpec((c0, n_nodes), lambda i: (0, 0)),
            pl.BlockSpec((c0, c0), lambda i: (0, 0)),
        ],
        out_specs=pl.BlockSpec((1, c0, 1), lambda i: (i, 0, 0)),
        out_shape=jax.ShapeDtypeStruct((g1, c0, 1), jnp.float32),
        scratch_shapes=[pltpu.VMEM((c0, n_nodes), jnp.float32)],
        compiler_params=params,
    )(neighborhood_0_to_0, x0at, w00_l1)

    s2p, s1 = pl.pallas_call(
        _stream3_kernel,
        grid=(g3,),
        in_specs=[
            _n_spec(bm, n_edges),
            pl.BlockSpec((n_edges, c1), lambda i: (0, 0)),
            pl.BlockSpec((c1, c2), lambda i: (0, 0)),
        ],
        out_specs=(
            pl.BlockSpec((1, c2, 1), lambda i: (i, 0, 0)),
            pl.BlockSpec((1, c1), lambda i: (0, 0)),
        ),
        out_shape=(
            jax.ShapeDtypeStruct((g3, c2, 1), jnp.float32),
            jax.ShapeDtypeStruct((1, c1), jnp.float32),
        ),
        scratch_shapes=[pltpu.VMEM((c2, n_edges), jnp.float32)],
        compiler_params=params,
    )(neighborhood_1_to_2, x_1, w12_l1)

    final = functools.partial(_final_kernel, n_nodes=float(n_nodes),
                              n_edges=float(n_edges), n_faces=float(n_faces))
    out = pl.pallas_call(
        final,
        out_shape=jax.ShapeDtypeStruct((1, ncls), jnp.float32),
    )(s0p, s2p, s1,
      lin0_w, lin0_b.reshape(1, ncls), lin1_w, lin1_b.reshape(1, ncls),
      lin2_w, lin2_b.reshape(1, ncls))
    return out.reshape(ncls)
